# baseline (speedup 1.0000x reference)
import functools

import jax
import jax.numpy as jnp
from jax import lax
from jax.experimental import pallas as pl
from jax.experimental.pallas import tpu as pltpu
from jax.experimental.pallas import tpu_sc as plsc

_B = 16384
_D = 32
_L = 16
_NC = 2
_NS = 16
_NW = _NC * _NS
_BPW = _B // _NW
_NG = _BPW // _L

_NBUF = 10


def _mf_body(users_hbm, items_hbm, ut_hbm, it_hbm, out_hbm,
             idx_u, idx_i, ubuf, vbuf, urows, vrows,
             out_v, sem_u, sem_v):
  wid = lax.axis_index("s") * _NC + lax.axis_index("c")
  base = wid * _BPW
  pltpu.sync_copy(users_hbm.at[pl.ds(base, _BPW)], idx_u.at[pl.ds(0, _BPW)])
  pltpu.sync_copy(items_hbm.at[pl.ds(base, _BPW)], idx_i.at[pl.ds(0, _BPW)])

  lane = lax.iota(jnp.int32, _L)

  def fire(ru, ri, slot):
    pltpu.async_copy(
        ut_hbm.at[:, pl.ds(pl.multiple_of((ru >> 7) << 7, 128), 128)],
        ubuf.at[slot], sem_u)
    pltpu.async_copy(
        it_hbm.at[:, pl.ds(pl.multiple_of((ri >> 7) << 7, 128), 128)],
        vbuf.at[slot], sem_v)

  u0 = idx_u[pl.ds(0, _L)]
  i0 = idx_i[pl.ds(0, _L)]
  for j in range(_NBUF):
    fire(u0[j], i0[j], j)

  def group(g, carry):
    cur_u = idx_u[pl.ds(pl.multiple_of(g * _L, _L), _L)]
    cur_i = idx_i[pl.ds(pl.multiple_of(g * _L, _L), _L)]
    nxt_u = idx_u[pl.ds(pl.multiple_of(g * _L + _L, _L), _L)]
    nxt_i = idx_i[pl.ds(pl.multiple_of(g * _L + _L, _L), _L)]
    u_lo = jnp.zeros((_L,), jnp.float32)
    u_hi = jnp.zeros((_L,), jnp.float32)
    v_lo = jnp.zeros((_L,), jnp.float32)
    v_hi = jnp.zeros((_L,), jnp.float32)
    accs = [u_lo, u_hi, v_lo, v_hi]

    for j in range(_L):
      k = g * _L + j
      slot = lax.rem(k, _NBUF)
      pltpu.make_async_copy(
          ut_hbm.at[:, pl.ds(pl.multiple_of(0, 128), 128)],
          ubuf.at[slot], sem_u).wait()
      pltpu.make_async_copy(
          it_hbm.at[:, pl.ds(pl.multiple_of(0, 128), 128)],
          vbuf.at[slot], sem_v).wait()
      ru = cur_u[j]
      ri = cur_i[j]
      ss = jnp.full((_L,), slot, jnp.int32)
      mu = jnp.full((_L,), ru & 127, jnp.int32)
      mi = jnp.full((_L,), ri & 127, jnp.int32)
      pos = pl.ds(pl.multiple_of(k * _D, _L), _L)
      pos_hi = pl.ds(pl.multiple_of(k * _D + _L, _L), _L)
      urows[pos] = plsc.load_gather(ubuf, [ss, lane, mu])
      urows[pos_hi] = plsc.load_gather(ubuf, [ss, lane + _L, mu])
      vrows[pos] = plsc.load_gather(vbuf, [ss, lane, mi])
      vrows[pos_hi] = plsc.load_gather(vbuf, [ss, lane + _L, mi])
      jn = j + _NBUF
      if jn < _L:
        rn_u, rn_i = cur_u[jn], cur_i[jn]
      else:
        rn_u, rn_i = nxt_u[jn - _L], nxt_i[jn - _L]

      @pl.when(k + _NBUF < _BPW)
      def _():
        fire(rn_u, rn_i, slot)

    return carry

  lax.fori_loop(0, _NG, group, 0)

  def dot_group(g, carry):
    pos0 = g * (_L * _D) + lane * _D
    acc = jnp.zeros((_L,), jnp.float32)
    for c in range(_D):
      pos = pos0 + c
      acc = acc + (plsc.load_gather(urows, [pos])
                   * plsc.load_gather(vrows, [pos]))
    r = 1.0 / (1.0 + jnp.exp(-acc))
    out_v[pl.ds(pl.multiple_of(g * _L, _L), _L)] = r
    return carry

  lax.fori_loop(0, _NG, dot_group, 0)
  pltpu.sync_copy(out_v, out_hbm.at[pl.ds(base, _BPW)])


_mf = functools.partial(
    pl.kernel,
    out_type=jax.ShapeDtypeStruct((_B,), jnp.float32),
    mesh=plsc.VectorSubcoreMesh(core_axis_name="c", subcore_axis_name="s"),
    scratch_types=[
        pltpu.VMEM((_BPW + _L,), jnp.int32),
        pltpu.VMEM((_BPW + _L,), jnp.int32),
        pltpu.VMEM((_NBUF, _D, 128), jnp.float32),
        pltpu.VMEM((_NBUF, _D, 128), jnp.float32),
        pltpu.VMEM((_BPW * _D,), jnp.float32),
        pltpu.VMEM((_BPW * _D,), jnp.float32),
        pltpu.VMEM((_BPW,), jnp.float32),
        pltpu.SemaphoreType.DMA,
        pltpu.SemaphoreType.DMA,
    ],
    compiler_params=pltpu.CompilerParams(
        needs_layout_passes=False, disable_bounds_checks=True),
)(_mf_body)


def kernel(users, items, user_table, item_table):
  out = _mf(users.astype(jnp.int32), items.astype(jnp.int32),
            user_table.T, item_table.T)
  return out.reshape(_B, 1)


# speedup vs baseline: 1.0035x; 1.0035x over previous
import functools

import jax
import jax.numpy as jnp
from jax import lax
from jax.experimental import pallas as pl
from jax.experimental.pallas import tpu as pltpu
from jax.experimental.pallas import tpu_sc as plsc

_B = 16384
_D = 32
_L = 16
_NC = 2
_NS = 16
_NW = _NC * _NS
_BPW = _B // _NW
_NG = _BPW // _L

_NBUF = 8


def _mf_body(users_hbm, items_hbm, ut_hbm, it_hbm, out_hbm,
             idx_u, idx_i, ubuf, vbuf, urows, vrows,
             out_v, sem_u, sem_v):
  wid = lax.axis_index("s") * _NC + lax.axis_index("c")
  base = wid * _BPW
  pltpu.sync_copy(users_hbm.at[pl.ds(base, _BPW)], idx_u.at[pl.ds(0, _BPW)])
  pltpu.sync_copy(items_hbm.at[pl.ds(base, _BPW)], idx_i.at[pl.ds(0, _BPW)])

  lane = lax.iota(jnp.int32, _L)

  def fire(ru, ri, slot):
    for t in range(4):
      pltpu.async_copy(
          ut_hbm.at[pl.ds(t * 8, 8),
                    pl.ds(pl.multiple_of((ru >> 7) << 7, 128), 128)],
          ubuf.at[slot, pl.ds(t * 8, 8)], sem_u)
      pltpu.async_copy(
          it_hbm.at[pl.ds(t * 8, 8),
                    pl.ds(pl.multiple_of((ri >> 7) << 7, 128), 128)],
          vbuf.at[slot, pl.ds(t * 8, 8)], sem_v)

  u0 = idx_u[pl.ds(0, _L)]
  i0 = idx_i[pl.ds(0, _L)]
  for j in range(_NBUF):
    fire(u0[j], i0[j], j)

  def group(g, carry):
    cur_u = idx_u[pl.ds(pl.multiple_of(g * _L, _L), _L)]
    cur_i = idx_i[pl.ds(pl.multiple_of(g * _L, _L), _L)]
    nxt_u = idx_u[pl.ds(pl.multiple_of(g * _L + _L, _L), _L)]
    nxt_i = idx_i[pl.ds(pl.multiple_of(g * _L + _L, _L), _L)]
    u_lo = jnp.zeros((_L,), jnp.float32)
    u_hi = jnp.zeros((_L,), jnp.float32)
    v_lo = jnp.zeros((_L,), jnp.float32)
    v_hi = jnp.zeros((_L,), jnp.float32)
    accs = [u_lo, u_hi, v_lo, v_hi]

    for j in range(_L):
      k = g * _L + j
      slot = lax.rem(k, _NBUF)
      for t in range(4):
        pltpu.make_async_copy(
            ut_hbm.at[pl.ds(t * 8, 8), pl.ds(pl.multiple_of(0, 128), 128)],
            ubuf.at[slot, pl.ds(t * 8, 8)], sem_u).wait()
        pltpu.make_async_copy(
            it_hbm.at[pl.ds(t * 8, 8), pl.ds(pl.multiple_of(0, 128), 128)],
            vbuf.at[slot, pl.ds(t * 8, 8)], sem_v).wait()
      ru = cur_u[j]
      ri = cur_i[j]
      ss = jnp.full((_L,), slot, jnp.int32)
      mu = jnp.full((_L,), ru & 127, jnp.int32)
      mi = jnp.full((_L,), ri & 127, jnp.int32)
      pos = pl.ds(pl.multiple_of(k * _D, _L), _L)
      pos_hi = pl.ds(pl.multiple_of(k * _D + _L, _L), _L)
      urows[pos] = plsc.load_gather(ubuf, [ss, lane, mu])
      urows[pos_hi] = plsc.load_gather(ubuf, [ss, lane + _L, mu])
      vrows[pos] = plsc.load_gather(vbuf, [ss, lane, mi])
      vrows[pos_hi] = plsc.load_gather(vbuf, [ss, lane + _L, mi])
      jn = j + _NBUF
      if jn < _L:
        rn_u, rn_i = cur_u[jn], cur_i[jn]
      else:
        rn_u, rn_i = nxt_u[jn - _L], nxt_i[jn - _L]

      @pl.when(k + _NBUF < _BPW)
      def _():
        fire(rn_u, rn_i, slot)

    return carry

  lax.fori_loop(0, _NG, group, 0)

  def dot_group(g, carry):
    pos0 = g * (_L * _D) + lane * _D
    acc = jnp.zeros((_L,), jnp.float32)
    for c in range(_D):
      pos = pos0 + c
      acc = acc + (plsc.load_gather(urows, [pos])
                   * plsc.load_gather(vrows, [pos]))
    r = 1.0 / (1.0 + jnp.exp(-acc))
    out_v[pl.ds(pl.multiple_of(g * _L, _L), _L)] = r
    return carry

  lax.fori_loop(0, _NG, dot_group, 0)
  pltpu.sync_copy(out_v, out_hbm.at[pl.ds(base, _BPW)])


_mf = functools.partial(
    pl.kernel,
    out_type=jax.ShapeDtypeStruct((_B,), jnp.float32),
    mesh=plsc.VectorSubcoreMesh(core_axis_name="c", subcore_axis_name="s"),
    scratch_types=[
        pltpu.VMEM((_BPW + _L,), jnp.int32),
        pltpu.VMEM((_BPW + _L,), jnp.int32),
        pltpu.VMEM((_NBUF, _D, 128), jnp.float32),
        pltpu.VMEM((_NBUF, _D, 128), jnp.float32),
        pltpu.VMEM((_BPW * _D,), jnp.float32),
        pltpu.VMEM((_BPW * _D,), jnp.float32),
        pltpu.VMEM((_BPW,), jnp.float32),
        pltpu.SemaphoreType.DMA,
        pltpu.SemaphoreType.DMA,
    ],
    compiler_params=pltpu.CompilerParams(
        needs_layout_passes=False, disable_bounds_checks=True),
)(_mf_body)


def kernel(users, items, user_table, item_table):
  out = _mf(users.astype(jnp.int32), items.astype(jnp.int32),
            user_table.T, item_table.T)
  return out.reshape(_B, 1)


# speedup vs baseline: 1.0104x; 1.0069x over previous
import functools

import jax
import jax.numpy as jnp
from jax import lax
from jax.experimental import pallas as pl
from jax.experimental.pallas import tpu as pltpu
from jax.experimental.pallas import tpu_sc as plsc

_B = 16384
_D = 32
_L = 16
_NC = 2
_NS = 16
_NW = _NC * _NS
_BPW = _B // _NW
_NG = _BPW // _L

_NBUF = 8


def _mf_body(users_hbm, items_hbm, ut_hbm, it_hbm, out_hbm,
             idx_u, idx_i, ubuf, vbuf, urows, vrows,
             out_v, sem_u, sem_v):
  wid = lax.axis_index("s") * _NC + lax.axis_index("c")
  base = wid * _BPW
  pltpu.sync_copy(users_hbm.at[pl.ds(base, _BPW)], idx_u.at[pl.ds(0, _BPW)])
  pltpu.sync_copy(items_hbm.at[pl.ds(base, _BPW)], idx_i.at[pl.ds(0, _BPW)])

  lane = lax.iota(jnp.int32, _L)

  def fire(ru, ri, slot):
    pltpu.async_copy(
        ut_hbm.at[:, pl.ds(pl.multiple_of((ru >> 7) << 7, 128), 128)],
        ubuf.at[slot], sem_u)
    pltpu.async_copy(
        it_hbm.at[:, pl.ds(pl.multiple_of((ri >> 7) << 7, 128), 128)],
        vbuf.at[slot], sem_v)

  u0 = idx_u[pl.ds(0, _L)]
  i0 = idx_i[pl.ds(0, _L)]
  for j in range(_NBUF):
    fire(u0[j], i0[j], j)

  def group(g, carry):
    cur_u = idx_u[pl.ds(pl.multiple_of(g * _L, _L), _L)]
    cur_i = idx_i[pl.ds(pl.multiple_of(g * _L, _L), _L)]
    nxt_u = idx_u[pl.ds(pl.multiple_of(g * _L + _L, _L), _L)]
    nxt_i = idx_i[pl.ds(pl.multiple_of(g * _L + _L, _L), _L)]
    u_lo = jnp.zeros((_L,), jnp.float32)
    u_hi = jnp.zeros((_L,), jnp.float32)
    v_lo = jnp.zeros((_L,), jnp.float32)
    v_hi = jnp.zeros((_L,), jnp.float32)
    accs = [u_lo, u_hi, v_lo, v_hi]

    for j in range(_L):
      k = g * _L + j
      slot = lax.rem(k, _NBUF)
      pltpu.make_async_copy(
          ut_hbm.at[:, pl.ds(pl.multiple_of(0, 128), 128)],
          ubuf.at[slot], sem_u).wait()
      pltpu.make_async_copy(
          it_hbm.at[:, pl.ds(pl.multiple_of(0, 128), 128)],
          vbuf.at[slot], sem_v).wait()
      ru = cur_u[j]
      ri = cur_i[j]
      ss = jnp.full((_L,), slot, jnp.int32)
      mu = jnp.full((_L,), ru & 127, jnp.int32)
      mi = jnp.full((_L,), ri & 127, jnp.int32)
      pos = pl.ds(pl.multiple_of(k * _D, _L), _L)
      pos_hi = pl.ds(pl.multiple_of(k * _D + _L, _L), _L)
      urows[pos] = plsc.load_gather(ubuf, [ss, lane, mu])
      urows[pos_hi] = plsc.load_gather(ubuf, [ss, lane + _L, mu])
      vrows[pos] = plsc.load_gather(vbuf, [ss, lane, mi])
      vrows[pos_hi] = plsc.load_gather(vbuf, [ss, lane + _L, mi])
      jn = j + _NBUF
      if jn < _L:
        rn_u, rn_i = cur_u[jn], cur_i[jn]
      else:
        rn_u, rn_i = nxt_u[jn - _L], nxt_i[jn - _L]

      @pl.when(k + _NBUF < _BPW)
      def _():
        fire(rn_u, rn_i, slot)

    return carry

  lax.fori_loop(0, _NG, group, 0)

  def dot_group(g, carry):
    pos0 = g * (_L * _D) + lane * _D
    acc = jnp.zeros((_L,), jnp.float32)
    for c in range(_D):
      pos = pos0 + c
      acc = acc + (plsc.load_gather(urows, [pos])
                   * plsc.load_gather(vrows, [pos]))
    r = 1.0 / (1.0 + jnp.exp(-acc))
    out_v[pl.ds(pl.multiple_of(g * _L, _L), _L)] = r
    return carry

  lax.fori_loop(0, _NG, dot_group, 0)
  pltpu.sync_copy(out_v, out_hbm.at[pl.ds(base, _BPW)])


_mf = functools.partial(
    pl.kernel,
    out_type=jax.ShapeDtypeStruct((_B,), jnp.float32),
    mesh=plsc.VectorSubcoreMesh(core_axis_name="c", subcore_axis_name="s"),
    scratch_types=[
        pltpu.VMEM((_BPW + _L,), jnp.int32),
        pltpu.VMEM((_BPW + _L,), jnp.int32),
        pltpu.VMEM((_NBUF, _D, 128), jnp.float32),
        pltpu.VMEM((_NBUF, _D, 128), jnp.float32),
        pltpu.VMEM((_BPW * _D,), jnp.float32),
        pltpu.VMEM((_BPW * _D,), jnp.float32),
        pltpu.VMEM((_BPW,), jnp.float32),
        pltpu.SemaphoreType.DMA,
        pltpu.SemaphoreType.DMA,
    ],
    compiler_params=pltpu.CompilerParams(
        needs_layout_passes=False, disable_bounds_checks=True),
)(_mf_body)


def kernel(users, items, user_table, item_table):
  out = _mf(users.astype(jnp.int32), items.astype(jnp.int32),
            user_table.T, item_table.T)
  return out.reshape(_B, 1)
